# Initial kernel scaffold; baseline (speedup 1.0000x reference)
import jax
import jax.numpy as jnp
from jax.experimental import pallas as pl


def kernel(z, W):
    raise NotImplementedError("write your pallas kernel here")


# speedup vs baseline: 1.1405x; 1.1405x over previous
import functools

import jax
import jax.numpy as jnp
from jax import lax
from jax.experimental import pallas as pl
from jax.experimental.pallas import tpu as pltpu
from jax.experimental.pallas import tpu_sc as plsc

B, D, T = 16, 256, 1024
K = 8192
NTOK = B * T
TB = 256
NBLK = NTOK // TB
COMMIT = 0.1

_SC_NC, _SC_NS = 2, 16
_NW = _SC_NC * _SC_NS
_RPW = NTOK // _NW
_CHUNK = 128
_NCH = _RPW // _CHUNK


def _dist_argmin_body(zp_ref, z2_ref, wt_ref, w2_ref, idx_ref, msum_ref):
    g = pl.program_id(0)
    zb = zp_ref[...]
    wt = wt_ref[...]
    zw = lax.dot_general(zb, wt, (((1,), (0,)), ((), ())),
                         preferred_element_type=jnp.float32)
    sv = (z2_ref[...] - zw) + w2_ref[...]
    msmall = jnp.min(sv, axis=1, keepdims=True)
    iota = lax.broadcasted_iota(jnp.int32, (TB, K), 1)
    masked = jnp.where(sv == msmall, iota, jnp.int32(2**30))
    idx_ref[0, 0, :] = jnp.min(masked, axis=1)

    @pl.when(g == 0)
    def _():
        msum_ref[...] = jnp.zeros((1, 1), jnp.float32)

    msum_ref[...] += jnp.sum(msmall, keepdims=True)


def _assemble_body(zq_ref, z_ref, out_ref):
    zq = zq_ref[...]
    zb = z_ref[0]
    zqt = zq.T
    out_ref[0] = zb + (zqt - zb)


def _sc_gather(Wf, idx_flat):
    idx3 = idx_flat.reshape(_NW, _NCH, _CHUNK)
    mesh = plsc.VectorSubcoreMesh(core_axis_name="c", subcore_axis_name="s")

    @functools.partial(
        pl.kernel,
        out_type=jax.ShapeDtypeStruct((NTOK, D), jnp.float32),
        mesh=mesh,
        scratch_types=[
            pltpu.VMEM((_NCH, _CHUNK), jnp.int32),
            pltpu.VMEM((2, _CHUNK, D), jnp.float32),
            pltpu.SemaphoreType.DMA,
            pltpu.SemaphoreType.DMA,
        ],
    )
    def gather_k(w_hbm, idx_hbm, out_hbm, idx_v, rows_v, sem0, sem1):
        wid = lax.axis_index("s") * _SC_NC + lax.axis_index("c")
        pltpu.sync_copy(idx_hbm.at[wid], idx_v)
        sems = (sem0, sem1)
        pending = {0: pltpu.async_copy(w_hbm.at[idx_v.at[0]], rows_v.at[0], sems[0])}
        for c in range(_NCH):
            if c + 1 < _NCH:
                pending[(c + 1) % 2] = pltpu.async_copy(
                    w_hbm.at[idx_v.at[c + 1]], rows_v.at[(c + 1) % 2],
                    sems[(c + 1) % 2])
            pending[c % 2].wait()
            pltpu.sync_copy(rows_v.at[c % 2],
                            out_hbm.at[pl.ds(wid * _RPW + c * _CHUNK, _CHUNK)])

    return gather_k(Wf, idx3)


def kernel(z, W):
    z_perm = jnp.transpose(z, (0, 2, 1)).reshape(NTOK, D)
    lhs = (2.0 * z_perm).astype(jnp.bfloat16)
    z2 = jnp.sum(z ** 2, axis=1).reshape(NTOK, 1)
    w2 = jnp.sum(W ** 2, axis=1)[None, :]
    wt = W.T

    idx_blocks, msum = pl.pallas_call(
        _dist_argmin_body,
        grid=(NBLK,),
        in_specs=[
            pl.BlockSpec((TB, D), lambda g: (g, 0)),
            pl.BlockSpec((TB, 1), lambda g: (g, 0)),
            pl.BlockSpec((D, K), lambda g: (0, 0)),
            pl.BlockSpec((1, K), lambda g: (0, 0)),
        ],
        out_specs=[
            pl.BlockSpec((1, 1, TB), lambda g: (g, 0, 0)),
            pl.BlockSpec((1, 1), lambda g: (0, 0)),
        ],
        out_shape=[
            jax.ShapeDtypeStruct((NBLK, 1, TB), jnp.int32),
            jax.ShapeDtypeStruct((1, 1), jnp.float32),
        ],
        compiler_params=pltpu.CompilerParams(
            dimension_semantics=("arbitrary",),
        ),
    )(lhs, z2, wt, w2)

    idx_flat = idx_blocks.reshape(NTOK)
    zq_rows = _sc_gather(W, idx_flat)

    z_q_st = pl.pallas_call(
        _assemble_body,
        grid=(NBLK,),
        in_specs=[
            pl.BlockSpec((TB, D), lambda g: (g, 0)),
            pl.BlockSpec((1, D, TB), lambda g: (g // (T // TB), 0, g % (T // TB))),
        ],
        out_specs=pl.BlockSpec((1, D, TB),
                               lambda g: (g // (T // TB), 0, g % (T // TB))),
        out_shape=jax.ShapeDtypeStruct((B, D, T), jnp.float32),
    )(zq_rows, z)

    indices = idx_flat.reshape(B, T)
    m_mean = msum[0, 0] / jnp.float32(NTOK * D)
    loss = m_mean + COMMIT * m_mean
    return z_q_st, indices, loss
